# baseline (speedup 1.0000x reference)
import functools

import jax
import jax.numpy as jnp
from jax import lax
from jax.experimental import pallas as pl
from jax.experimental.pallas import tpu as pltpu
from jax.experimental.pallas import tpu_sc as plsc

N = 10000
E = 320000
D = 128
NRBF = 20
CUTOFF = 5.0
NI = 3
MAX_Z = 100

MZP = 104
NRBFP = 24
LOG2 = 0.6931471805599453

NCORES = 2
NSUB = 16
NW = NCORES * NSUB
CHUNK = 64
NCHUNKS = 5056
EPAD = NCHUNKS * CHUNK
C0CH = 194
C1CH = 122
NPAD = 10112
STRIPE = NPAD // NSUB
BN = 1000
BE = 512


def _ssp(x):
    return jnp.maximum(x, 0.0) + jnp.log(1.0 + jnp.exp(-jnp.abs(x))) - LOG2


def _embed_body(z_ref, emb_ref, out_ref):
    z = z_ref[...]
    col = lax.broadcasted_iota(jnp.int32, (BN, MZP), 1)
    oh = (z == col).astype(jnp.float32)
    out_ref[...] = jnp.dot(oh, emb_ref[...], preferred_element_type=jnp.float32)


def _embed(z2, embp):
    return pl.pallas_call(
        _embed_body,
        grid=(N // BN,),
        in_specs=[
            pl.BlockSpec((BN, 1), lambda i: (i, 0)),
            pl.BlockSpec((MZP, D), lambda i: (0, 0)),
        ],
        out_specs=pl.BlockSpec((BN, D), lambda i: (i, 0)),
        out_shape=jax.ShapeDtypeStruct((N, D), jnp.float32),
    )(z2, embp)


def _mm_body(x_ref, w_ref, out_ref):
    out_ref[...] = jnp.dot(x_ref[...], w_ref[...], preferred_element_type=jnp.float32)


def _in2f(x, w):
    return pl.pallas_call(
        _mm_body,
        grid=(N // BN,),
        in_specs=[
            pl.BlockSpec((BN, D), lambda i: (i, 0)),
            pl.BlockSpec((D, D), lambda i: (0, 0)),
        ],
        out_specs=pl.BlockSpec((BN, D), lambda i: (i, 0)),
        out_shape=jax.ShapeDtypeStruct((N, D), jnp.float32),
    )(x, w)


def _filter_body(r_ref, wf1_ref, bf1_ref, wf2_ref, bf2_ref, out_ref):
    i = pl.program_id(0)

    @pl.when(i < E // BE)
    def _real():
        r = r_ref[...]
        d2 = jnp.sum(r * r, axis=1, keepdims=True)
        d = jnp.sqrt(d2)
        delta = CUTOFF / (NRBF - 1)
        offs = delta * lax.broadcasted_iota(
            jnp.int32, (1, NRBFP), 1).astype(jnp.float32)
        coeff = -0.5 / (delta * delta)
        f = jnp.exp(coeff * (d - offs) ** 2)
        h = _ssp(jnp.dot(f, wf1_ref[...], preferred_element_type=jnp.float32)
                 + bf1_ref[...])
        w = jnp.dot(h, wf2_ref[...], preferred_element_type=jnp.float32) \
            + bf2_ref[...]
        y = jnp.minimum(d * (jnp.pi / CUTOFF), jnp.pi)
        u = y * y
        c = jnp.float32(1.0 / 20922789888000.0)
        for k, fac in ((14, 87178291200.0), (12, 479001600.0), (10, 3628800.0),
                       (8, 40320.0), (6, 720.0), (4, 24.0), (2, 2.0)):
            sign = -1.0 if (k // 2) % 2 else 1.0
            c = c * u + jnp.float32(sign / fac)
        cosy = c * u + 1.0
        rcut = 0.5 * (cosy + 1.0)
        rcut = rcut * (d < CUTOFF).astype(jnp.float32)
        out_ref[...] = w * rcut

    @pl.when(i >= E // BE)
    def _pad():
        out_ref[...] = jnp.zeros((BE, D), jnp.float32)


def _filter(rij, wf1p, bf1, wf2, bf2):
    return pl.pallas_call(
        _filter_body,
        grid=(EPAD // BE,),
        in_specs=[
            pl.BlockSpec((BE, 3), lambda i: (jnp.minimum(i, E // BE - 1), 0)),
            pl.BlockSpec((NRBFP, D), lambda i: (0, 0)),
            pl.BlockSpec((1, D), lambda i: (0, 0)),
            pl.BlockSpec((D, D), lambda i: (0, 0)),
            pl.BlockSpec((1, D), lambda i: (0, 0)),
        ],
        out_specs=pl.BlockSpec((BE, D), lambda i: (i, 0)),
        out_shape=jax.ShapeDtypeStruct((EPAD, D), jnp.float32),
    )(rij, wf1p, bf1, wf2, bf2)


def _out_body(agg_ref, x_ref, w1_ref, b1_ref, w2_ref, b2_ref, out_ref):
    agg = agg_ref[0] + agg_ref[1]
    h = _ssp(jnp.dot(agg, w1_ref[...], preferred_element_type=jnp.float32)
             + b1_ref[...])
    v = jnp.dot(h, w2_ref[...], preferred_element_type=jnp.float32) + b2_ref[...]
    out_ref[...] = x_ref[...] + v


def _out(agg_p, x, w1, b1, w2, b2):
    return pl.pallas_call(
        _out_body,
        grid=(N // BN,),
        in_specs=[
            pl.BlockSpec((2, BN, D), lambda i: (0, i, 0)),
            pl.BlockSpec((BN, D), lambda i: (i, 0)),
            pl.BlockSpec((D, D), lambda i: (0, 0)),
            pl.BlockSpec((1, D), lambda i: (0, 0)),
            pl.BlockSpec((D, D), lambda i: (0, 0)),
            pl.BlockSpec((1, D), lambda i: (0, 0)),
        ],
        out_specs=pl.BlockSpec((BN, D), lambda i: (i, 0)),
        out_shape=jax.ShapeDtypeStruct((N, D), jnp.float32),
    )(agg_p, x, w1, b1, w2, b2)


def _sc_edge_body(xf_hbm, wij_hbm, idxi_hbm, idxj_hbm, out_hbm,
                  idxi0, idxi1, idxj0, idxj1, rows0, rows1, wij0, wij1,
                  agg_sh, ii0, ii1, ij0, ij1, g0, g1, w0, w1, s0, s1):
    cid = lax.axis_index("c")
    sid = lax.axis_index("s")
    cbase = jnp.where(cid == 0, sid * C0CH, NSUB * C0CH + sid * C1CH)
    nch = jnp.where(cid == 0, C0CH, C1CH)

    zeros16 = jnp.zeros((16,), jnp.float32)

    @plsc.parallel_loop(0, CHUNK, unroll=4)
    def _zero_row(e):
        for k in range(D // 16):
            wij0[e, pl.ds(k * 16, 16)] = zeros16
    for t in range(STRIPE // CHUNK):
        pltpu.sync_copy(wij0, agg_sh.at[pl.ds(sid * STRIPE + t * CHUNK, CHUNK)])
    rem = STRIPE - (STRIPE // CHUNK) * CHUNK
    if rem:
        pltpu.sync_copy(wij0.at[pl.ds(0, rem)],
                        agg_sh.at[pl.ds(sid * STRIPE + STRIPE - rem, rem)])
    plsc.subcore_barrier()

    def _start(t, idxi_v, idxj_v, isi, isj):
        base = (cbase + t) * CHUNK
        pltpu.async_copy(idxi_hbm.at[pl.ds(base, CHUNK)], idxi_v, isi)
        pltpu.async_copy(idxj_hbm.at[pl.ds(base, CHUNK)], idxj_v, isj)

    def _mid(t, idxi_v, idxj_v, rows_v, wij_v, isi, isj, g, w):
        base = (cbase + t) * CHUNK
        pltpu.make_async_copy(idxi_hbm.at[pl.ds(base, CHUNK)], idxi_v, isi).wait()
        pltpu.make_async_copy(idxj_hbm.at[pl.ds(base, CHUNK)], idxj_v, isj).wait()
        pltpu.async_copy(xf_hbm.at[idxj_v], rows_v, g)
        pltpu.async_copy(wij_hbm.at[pl.ds(base, CHUNK)], wij_v, w)

    def _finish(t, idxi_v, idxj_v, rows_v, wij_v, g, w, s):
        base = (cbase + t) * CHUNK
        pltpu.make_async_copy(xf_hbm.at[idxj_v], rows_v, g).wait()
        pltpu.make_async_copy(
            wij_hbm.at[pl.ds(base, CHUNK)], wij_v, w).wait()

        @plsc.parallel_loop(0, CHUNK, unroll=4)
        def _mul(e):
            for k in range(D // 16):
                sl = pl.ds(k * 16, 16)
                rows_v[e, sl] = rows_v[e, sl] * wij_v[e, sl]

        pltpu.async_copy(rows_v, agg_sh.at[idxi_v], s, add=True)

    def _wait_s(idxi_v, rows_v, s):
        pltpu.make_async_copy(rows_v, agg_sh.at[idxi_v], s).wait()

    _start(0, idxi0, idxj0, ii0, ij0)
    _mid(0, idxi0, idxj0, rows0, wij0, ii0, ij0, g0, w0)
    _start(1, idxi1, idxj1, ii1, ij1)

    def _pair(p, _):
        t = 2 * p
        _mid(t + 1, idxi1, idxj1, rows1, wij1, ii1, ij1, g1, w1)
        _finish(t, idxi0, idxj0, rows0, wij0, g0, w0, s0)
        _wait_s(idxi0, rows0, s0)
        _start(t + 2, idxi0, idxj0, ii0, ij0)
        _finish(t + 1, idxi1, idxj1, rows1, wij1, g1, w1, s1)
        _wait_s(idxi1, rows1, s1)
        _start(t + 3, idxi1, idxj1, ii1, ij1)
        _mid(t + 2, idxi0, idxj0, rows0, wij0, ii0, ij0, g0, w0)
        return ()

    lax.fori_loop(0, nch // 2 - 1, _pair, ())
    _mid(nch - 1, idxi1, idxj1, rows1, wij1, ii1, ij1, g1, w1)
    _finish(nch - 2, idxi0, idxj0, rows0, wij0, g0, w0, s0)
    _wait_s(idxi0, rows0, s0)
    _finish(nch - 1, idxi1, idxj1, rows1, wij1, g1, w1, s1)
    _wait_s(idxi1, rows1, s1)

    plsc.subcore_barrier()
    pltpu.sync_copy(agg_sh.at[pl.ds(sid * STRIPE, STRIPE)],
                    out_hbm.at[cid, pl.ds(sid * STRIPE, STRIPE)])


_sc_edge_built = None


def _sc_edge(xf, wij, idx_i_p, idx_j_p):
    global _sc_edge_built
    if _sc_edge_built is None:
        mesh = plsc.VectorSubcoreMesh(core_axis_name="c", subcore_axis_name="s")
        _sc_edge_built = pl.kernel(
            _sc_edge_body,
            mesh=mesh,
            out_type=jax.ShapeDtypeStruct((NCORES, NPAD, D), jnp.float32),
            scratch_types=[
                pltpu.VMEM((CHUNK,), jnp.int32),
                pltpu.VMEM((CHUNK,), jnp.int32),
                pltpu.VMEM((CHUNK,), jnp.int32),
                pltpu.VMEM((CHUNK,), jnp.int32),
                pltpu.VMEM((CHUNK, D), jnp.float32),
                pltpu.VMEM((CHUNK, D), jnp.float32),
                pltpu.VMEM((CHUNK, D), jnp.float32),
                pltpu.VMEM((CHUNK, D), jnp.float32),
                pltpu.VMEM_SHARED((NPAD, D), jnp.float32),
            ] + [pltpu.SemaphoreType.DMA] * 10,
        )
    return _sc_edge_built(xf, wij, idx_i_p, idx_j_p)


def kernel(Z, Rij, idx_i, idx_j, emb, Win2f, Wf1, bf1, Wf2, bf2, Wo1, bo1, Wo2, bo2):
    embp = jnp.zeros((MZP, D), jnp.float32).at[:MAX_Z].set(emb)
    x = _embed(Z.reshape(N, 1).astype(jnp.int32), embp)
    npad_e = EPAD - E
    idx_i_p = jnp.concatenate(
        [idx_i.astype(jnp.int32), jnp.zeros((npad_e,), jnp.int32)])
    idx_j_p = jnp.concatenate(
        [idx_j.astype(jnp.int32), jnp.zeros((npad_e,), jnp.int32)])
    wijs = []
    for i in range(NI):
        wf1p = jnp.zeros((NRBFP, D), jnp.float32).at[:NRBF].set(Wf1[i])
        wijs.append(_filter(Rij, wf1p, bf1[i][None], Wf2[i], bf2[i][None]))
    for i in range(NI):
        xf = _in2f(x, Win2f[i])
        agg_p = _sc_edge(xf, wijs[i], idx_i_p, idx_j_p)
        x = _out(agg_p, x, Wo1[i], bo1[i][None], Wo2[i], bo2[i][None])
    return x


# speedup vs baseline: 1.2527x; 1.2527x over previous
import functools

import jax
import jax.numpy as jnp
from jax import lax
from jax.experimental import pallas as pl
from jax.experimental.pallas import tpu as pltpu
from jax.experimental.pallas import tpu_sc as plsc

N = 10000
E = 320000
D = 128
NRBF = 20
CUTOFF = 5.0
NI = 3
MAX_Z = 100

MZP = 104
NRBFP = 24
LOG2 = 0.6931471805599453

NCORES = 2
NSUB = 16
NW = NCORES * NSUB
CHUNK = 64
NCHUNKS = 5056
EPAD = NCHUNKS * CHUNK
C0CH = 194
C1CH = 122
NPAD = 10112
STRIPE = NPAD // NSUB
DUMP = NPAD - 1
BN = 1000
BE = 1000


def _ssp(x):
    return jnp.maximum(x, 0.0) + jnp.log(1.0 + jnp.exp(-jnp.abs(x))) - LOG2


def _embed_body(z_ref, emb_ref, out_ref):
    z = z_ref[...]
    col = lax.broadcasted_iota(jnp.int32, (BN, MZP), 1)
    oh = (z == col).astype(jnp.float32)
    out_ref[...] = jnp.dot(oh, emb_ref[...], preferred_element_type=jnp.float32)


def _embed(z2, embp):
    return pl.pallas_call(
        _embed_body,
        grid=(N // BN,),
        in_specs=[
            pl.BlockSpec((BN, 1), lambda i: (i, 0)),
            pl.BlockSpec((MZP, D), lambda i: (0, 0)),
        ],
        out_specs=pl.BlockSpec((BN, D), lambda i: (i, 0)),
        out_shape=jax.ShapeDtypeStruct((N, D), jnp.float32),
    )(z2, embp)


def _mm_body(x_ref, w_ref, out_ref):
    out_ref[...] = jnp.dot(x_ref[...], w_ref[...], preferred_element_type=jnp.float32)


def _in2f(x, w):
    return pl.pallas_call(
        _mm_body,
        grid=(N // BN,),
        in_specs=[
            pl.BlockSpec((BN, D), lambda i: (i, 0)),
            pl.BlockSpec((D, D), lambda i: (0, 0)),
        ],
        out_specs=pl.BlockSpec((BN, D), lambda i: (i, 0)),
        out_shape=jax.ShapeDtypeStruct((N, D), jnp.float32),
    )(x, w)


def _filter_body(r_ref, wf1_ref, bf1_ref, wf2_ref, bf2_ref, out_ref):
    r = r_ref[...]
    d2 = jnp.sum(r * r, axis=1, keepdims=True)
    d = jnp.sqrt(d2)
    delta = CUTOFF / (NRBF - 1)
    offs = delta * lax.broadcasted_iota(
        jnp.int32, (1, NRBFP), 1).astype(jnp.float32)
    coeff = -0.5 / (delta * delta)
    f = jnp.exp(coeff * (d - offs) ** 2)
    h = _ssp(jnp.dot(f, wf1_ref[...], preferred_element_type=jnp.float32)
             + bf1_ref[...])
    w = jnp.dot(h, wf2_ref[...], preferred_element_type=jnp.float32) \
        + bf2_ref[...]
    y = jnp.minimum(d * (jnp.pi / CUTOFF), jnp.pi)
    u = y * y
    c = jnp.float32(1.0 / 20922789888000.0)
    for k, fac in ((14, 87178291200.0), (12, 479001600.0), (10, 3628800.0),
                   (8, 40320.0), (6, 720.0), (4, 24.0), (2, 2.0)):
        sign = -1.0 if (k // 2) % 2 else 1.0
        c = c * u + jnp.float32(sign / fac)
    cosy = c * u + 1.0
    rcut = 0.5 * (cosy + 1.0)
    rcut = rcut * (d < CUTOFF).astype(jnp.float32)
    out_ref[...] = w * rcut


def _filter(rij, wf1p, bf1, wf2, bf2):
    return pl.pallas_call(
        _filter_body,
        grid=(E // BE,),
        in_specs=[
            pl.BlockSpec((BE, 3), lambda i: (i, 0)),
            pl.BlockSpec((NRBFP, D), lambda i: (0, 0)),
            pl.BlockSpec((1, D), lambda i: (0, 0)),
            pl.BlockSpec((D, D), lambda i: (0, 0)),
            pl.BlockSpec((1, D), lambda i: (0, 0)),
        ],
        out_specs=pl.BlockSpec((BE, D), lambda i: (i, 0)),
        out_shape=jax.ShapeDtypeStruct((E, D), jnp.float32),
    )(rij, wf1p, bf1, wf2, bf2)


def _out_body(agg_ref, x_ref, w1_ref, b1_ref, w2_ref, b2_ref, out_ref):
    agg = agg_ref[0] + agg_ref[1]
    h = _ssp(jnp.dot(agg, w1_ref[...], preferred_element_type=jnp.float32)
             + b1_ref[...])
    v = jnp.dot(h, w2_ref[...], preferred_element_type=jnp.float32) + b2_ref[...]
    out_ref[...] = x_ref[...] + v


def _out(agg_p, x, w1, b1, w2, b2):
    return pl.pallas_call(
        _out_body,
        grid=(N // BN,),
        in_specs=[
            pl.BlockSpec((2, BN, D), lambda i: (0, i, 0)),
            pl.BlockSpec((BN, D), lambda i: (i, 0)),
            pl.BlockSpec((D, D), lambda i: (0, 0)),
            pl.BlockSpec((1, D), lambda i: (0, 0)),
            pl.BlockSpec((D, D), lambda i: (0, 0)),
            pl.BlockSpec((1, D), lambda i: (0, 0)),
        ],
        out_specs=pl.BlockSpec((BN, D), lambda i: (i, 0)),
        out_shape=jax.ShapeDtypeStruct((N, D), jnp.float32),
    )(agg_p, x, w1, b1, w2, b2)


def _sc_edge_body(xf_hbm, wij_hbm, idxi_hbm, idxj_hbm, out_hbm,
                  idxi0, idxi1, idxj0, idxj1, rows0, rows1, wij0, wij1,
                  agg_sh, ii0, ii1, ij0, ij1, g0, g1, w0, w1, s0, s1):
    cid = lax.axis_index("c")
    sid = lax.axis_index("s")
    cbase = jnp.where(cid == 0, sid * C0CH, NSUB * C0CH + sid * C1CH)
    nch = jnp.where(cid == 0, C0CH, C1CH)

    zeros16 = jnp.zeros((16,), jnp.float32)

    @plsc.parallel_loop(0, CHUNK, unroll=4)
    def _zero_row(e):
        for k in range(D // 16):
            wij0[e, pl.ds(k * 16, 16)] = zeros16
    for t in range(STRIPE // CHUNK):
        pltpu.sync_copy(wij0, agg_sh.at[pl.ds(sid * STRIPE + t * CHUNK, CHUNK)])
    rem = STRIPE - (STRIPE // CHUNK) * CHUNK
    if rem:
        pltpu.sync_copy(wij0.at[pl.ds(0, rem)],
                        agg_sh.at[pl.ds(sid * STRIPE + STRIPE - rem, rem)])
    plsc.subcore_barrier()

    def _start(t, idxi_v, idxj_v, isi, isj):
        base = (cbase + t) * CHUNK
        pltpu.async_copy(idxi_hbm.at[pl.ds(base, CHUNK)], idxi_v, isi)
        pltpu.async_copy(idxj_hbm.at[pl.ds(base, CHUNK)], idxj_v, isj)

    def _mid(t, idxi_v, idxj_v, rows_v, wij_v, isi, isj, g, w):
        base = (cbase + t) * CHUNK
        wbase = jnp.minimum(base, E - CHUNK)
        pltpu.make_async_copy(idxi_hbm.at[pl.ds(base, CHUNK)], idxi_v, isi).wait()
        pltpu.make_async_copy(idxj_hbm.at[pl.ds(base, CHUNK)], idxj_v, isj).wait()
        pltpu.async_copy(xf_hbm.at[idxj_v], rows_v, g)
        pltpu.async_copy(wij_hbm.at[pl.ds(wbase, CHUNK)], wij_v, w)

    def _finish(t, idxi_v, idxj_v, rows_v, wij_v, g, w, s):
        base = (cbase + t) * CHUNK
        wbase = jnp.minimum(base, E - CHUNK)
        pltpu.make_async_copy(xf_hbm.at[idxj_v], rows_v, g).wait()
        pltpu.make_async_copy(
            wij_hbm.at[pl.ds(wbase, CHUNK)], wij_v, w).wait()

        @plsc.parallel_loop(0, CHUNK, unroll=4)
        def _mul(e):
            for k in range(D // 16):
                sl = pl.ds(k * 16, 16)
                rows_v[e, sl] = rows_v[e, sl] * wij_v[e, sl]

        pltpu.async_copy(rows_v, agg_sh.at[idxi_v], s, add=True)

    def _wait_s(idxi_v, rows_v, s):
        pltpu.make_async_copy(rows_v, agg_sh.at[idxi_v], s).wait()

    _start(0, idxi0, idxj0, ii0, ij0)
    _mid(0, idxi0, idxj0, rows0, wij0, ii0, ij0, g0, w0)
    _start(1, idxi1, idxj1, ii1, ij1)

    def _pair(p, _):
        t = 2 * p
        _mid(t + 1, idxi1, idxj1, rows1, wij1, ii1, ij1, g1, w1)
        _finish(t, idxi0, idxj0, rows0, wij0, g0, w0, s0)
        _wait_s(idxi0, rows0, s0)
        _start(t + 2, idxi0, idxj0, ii0, ij0)
        _finish(t + 1, idxi1, idxj1, rows1, wij1, g1, w1, s1)
        _wait_s(idxi1, rows1, s1)
        _start(t + 3, idxi1, idxj1, ii1, ij1)
        _mid(t + 2, idxi0, idxj0, rows0, wij0, ii0, ij0, g0, w0)
        return ()

    lax.fori_loop(0, nch // 2 - 1, _pair, ())
    _mid(nch - 1, idxi1, idxj1, rows1, wij1, ii1, ij1, g1, w1)
    _finish(nch - 2, idxi0, idxj0, rows0, wij0, g0, w0, s0)
    _wait_s(idxi0, rows0, s0)
    _finish(nch - 1, idxi1, idxj1, rows1, wij1, g1, w1, s1)
    _wait_s(idxi1, rows1, s1)

    plsc.subcore_barrier()
    pltpu.sync_copy(agg_sh.at[pl.ds(sid * STRIPE, STRIPE)],
                    out_hbm.at[cid, pl.ds(sid * STRIPE, STRIPE)])


_sc_edge_built = None


def _sc_edge(xf, wij, idx_i_p, idx_j_p):
    global _sc_edge_built
    if _sc_edge_built is None:
        mesh = plsc.VectorSubcoreMesh(core_axis_name="c", subcore_axis_name="s")
        _sc_edge_built = pl.kernel(
            _sc_edge_body,
            mesh=mesh,
            out_type=jax.ShapeDtypeStruct((NCORES, NPAD, D), jnp.float32),
            scratch_types=[
                pltpu.VMEM((CHUNK,), jnp.int32),
                pltpu.VMEM((CHUNK,), jnp.int32),
                pltpu.VMEM((CHUNK,), jnp.int32),
                pltpu.VMEM((CHUNK,), jnp.int32),
                pltpu.VMEM((CHUNK, D), jnp.float32),
                pltpu.VMEM((CHUNK, D), jnp.float32),
                pltpu.VMEM((CHUNK, D), jnp.float32),
                pltpu.VMEM((CHUNK, D), jnp.float32),
                pltpu.VMEM_SHARED((NPAD, D), jnp.float32),
            ] + [pltpu.SemaphoreType.DMA] * 10,
        )
    return _sc_edge_built(xf, wij, idx_i_p, idx_j_p)


def kernel(Z, Rij, idx_i, idx_j, emb, Win2f, Wf1, bf1, Wf2, bf2, Wo1, bo1, Wo2, bo2):
    embp = jnp.zeros((MZP, D), jnp.float32).at[:MAX_Z].set(emb)
    x = _embed(Z.reshape(N, 1).astype(jnp.int32), embp)
    npad_e = EPAD - E
    idx_i_p = jnp.concatenate(
        [idx_i.astype(jnp.int32), jnp.full((npad_e,), DUMP, jnp.int32)])
    idx_j_p = jnp.concatenate(
        [idx_j.astype(jnp.int32), jnp.zeros((npad_e,), jnp.int32)])
    wijs = []
    for i in range(NI):
        wf1p = jnp.zeros((NRBFP, D), jnp.float32).at[:NRBF].set(Wf1[i])
        wijs.append(_filter(Rij, wf1p, bf1[i][None], Wf2[i], bf2[i][None]))
    for i in range(NI):
        xf = _in2f(x, Win2f[i])
        agg_p = _sc_edge(xf, wijs[i], idx_i_p, idx_j_p)
        x = _out(agg_p, x, Wo1[i], bo1[i][None], Wo2[i], bo2[i][None])
    return x
